# baseline (speedup 1.0000x reference)
import dataclasses
import functools

import jax
import jax.numpy as jnp
from jax import lax
from jax.experimental import pallas as pl
from jax.experimental.pallas import tpu as pltpu
from jax.experimental.pallas import tpu_sc as plsc

DIM = 2048
INTER = 1408
E = 8
K = 2
T = 2048
A = T * K
MT = 256
P = A + E * MT
NT = P // MT
TT = 4
TM = T // TT

_NW = 32


def _sc_mesh():
    return plsc.VectorSubcoreMesh(core_axis_name="c", subcore_axis_name="s")


def _sc_params():
    cp = pltpu.CompilerParams()
    if "needs_layout_passes" in pltpu.CompilerParams.__dataclass_fields__:
        cp = dataclasses.replace(cp, needs_layout_passes=False)
    return cp


def _router_body(x_ref, wg_ref, d_ref, g_ref, eid_ref, act_ref):
    logits = jnp.dot(x_ref[...], wg_ref[...].T,
                     preferred_element_type=jnp.float32)
    eidx = lax.broadcasted_iota(jnp.int32, (T, E), 1)
    m1 = jnp.max(logits, axis=1, keepdims=True)
    e1 = jnp.min(jnp.where(logits >= m1, eidx, E), axis=1, keepdims=True)
    l2 = jnp.where(eidx == e1, -1e30, logits)
    m2 = jnp.max(l2, axis=1, keepdims=True)
    e2 = jnp.min(jnp.where(l2 >= m2, eidx, E), axis=1, keepdims=True)
    q = jnp.exp(m2 - m1)
    wa = 1.0 / (1.0 + q)
    wb = 1.0 - wa
    g_ref[...] = jnp.concatenate([wa, wb], axis=1)

    mask1 = (eidx == e1).astype(jnp.float32)
    mask2 = (eidx == e2).astype(jnp.float32)
    m = mask1 + mask2
    tri = (lax.broadcasted_iota(jnp.int32, (128, 128), 0)
           >= lax.broadcasted_iota(jnp.int32, (128, 128), 1)
           ).astype(jnp.float32)
    cex_blocks = []
    base = jnp.zeros((1, E), jnp.float32)
    for i in range(T // 128):
        mi = m[i * 128:(i + 1) * 128]
        inc = jnp.dot(tri, mi, preferred_element_type=jnp.float32)
        cex_blocks.append(inc - mi + base)
        base = base + inc[127:128, :]
    cex = jnp.concatenate(cex_blocks, axis=0)
    counts = base

    counts_i = counts.astype(jnp.int32)
    padded = ((counts_i + (MT - 1)) // MT) * MT
    stri = (lax.broadcasted_iota(jnp.int32, (E, E), 0)
            < lax.broadcasted_iota(jnp.int32, (E, E), 1)
            ).astype(jnp.float32)
    poffs = jnp.dot(padded.astype(jnp.float32), stri,
                    preferred_element_type=jnp.float32)

    slot = cex + poffs
    d0 = jnp.sum(jnp.where(eidx == e1, slot, 0.0), axis=1, keepdims=True)
    d1 = jnp.sum(jnp.where(eidx == e2, slot, 0.0), axis=1, keepdims=True)
    d_ref[...] = jnp.concatenate([d0, d1], axis=1).astype(jnp.int32)

    pend = poffs + padded.astype(jnp.float32)
    jstart = (lax.broadcasted_iota(jnp.int32, (NT, 1), 0) * MT
              ).astype(jnp.float32)
    eid = jnp.sum((jstart >= pend).astype(jnp.int32), axis=1, keepdims=True)
    eid_ref[...] = jnp.minimum(eid, E - 1)
    act_ref[...] = (jstart < pend[:, E - 1:E]).astype(jnp.int32)


def _router(xf, Wg):
    return pl.pallas_call(
        _router_body,
        out_shape=[
            jax.ShapeDtypeStruct((T, K), jnp.int32),
            jax.ShapeDtypeStruct((T, K), jnp.float32),
            jax.ShapeDtypeStruct((NT, 1), jnp.int32),
            jax.ShapeDtypeStruct((NT, 1), jnp.int32),
        ],
    )(xf, Wg)


def _dispatch_body(xb_hbm, dflat_hbm, xs_hbm, d_v, buf_v):
    wid = lax.axis_index("s") * 2 + lax.axis_index("c")
    a0 = wid * (A // _NW)

    @pl.loop(0, A // _NW // 16)
    def _(c):
        abase = a0 + c * 16
        pltpu.sync_copy(dflat_hbm.at[pl.ds(abase, 16)], d_v)
        tok = (abase + lax.iota(jnp.int32, 16)) >> 1
        pltpu.sync_copy(xb_hbm.at[tok], buf_v)
        pltpu.sync_copy(buf_v, xs_hbm.at[d_v])


@functools.cache
def _dispatch_kernel():
    return pl.kernel(
        _dispatch_body,
        out_type=jax.ShapeDtypeStruct((P, DIM // 2), jnp.int32),
        mesh=_sc_mesh(),
        scratch_types=[
            pltpu.VMEM((16,), jnp.int32),
            pltpu.VMEM((16, DIM // 2), jnp.int32),
        ],
        compiler_params=_sc_params(),
    )


def _gmm_body(eid_ref, act_ref, xs_ref, w1_ref, w3_ref, w2_ref, o_ref):
    j = pl.program_id(0)

    @pl.when(act_ref[j, 0] == 1)
    def _():
        xb = xs_ref[...]
        a = jnp.dot(xb, w1_ref[0].T, preferred_element_type=jnp.float32)
        b = jnp.dot(xb, w3_ref[0].T, preferred_element_type=jnp.float32)
        h = (jax.nn.silu(a) * b).astype(jnp.bfloat16)
        o_ref[...] = jnp.dot(h, w2_ref[0].T,
                             preferred_element_type=jnp.float32)


def _gmm(xs, w1b, w3b, w2b, eid, act):
    grid_spec = pltpu.PrefetchScalarGridSpec(
        num_scalar_prefetch=2,
        grid=(NT,),
        in_specs=[
            pl.BlockSpec((MT, DIM), lambda j, eid, act: (j, 0)),
            pl.BlockSpec((1, INTER, DIM),
                         lambda j, eid, act: (eid[j, 0], 0, 0)),
            pl.BlockSpec((1, INTER, DIM),
                         lambda j, eid, act: (eid[j, 0], 0, 0)),
            pl.BlockSpec((1, DIM, INTER),
                         lambda j, eid, act: (eid[j, 0], 0, 0)),
        ],
        out_specs=pl.BlockSpec((MT, DIM), lambda j, eid, act: (j, 0)),
    )
    return pl.pallas_call(
        _gmm_body,
        grid_spec=grid_spec,
        out_shape=jax.ShapeDtypeStruct((P, DIM), jnp.float32),
    )(eid, act, xs, w1b, w3b, w2b)


def _shared_body(xb_ref, ws1_ref, ws2_ref, o_ref):
    a = jnp.dot(xb_ref[...], ws1_ref[...].T,
                preferred_element_type=jnp.float32)
    g = (0.5 * a * (1.0 + lax.erf(a * 0.7071067811865476))).astype(
        jnp.bfloat16)
    o_ref[...] = jnp.dot(g, ws2_ref[...].T,
                         preferred_element_type=jnp.float32)


def _shared(xb, ws1b, ws2b):
    return pl.pallas_call(
        _shared_body,
        grid=(TT,),
        in_specs=[
            pl.BlockSpec((TM, DIM), lambda t: (t, 0)),
            pl.BlockSpec((INTER, DIM), lambda t: (0, 0)),
            pl.BlockSpec((DIM, INTER), lambda t: (0, 0)),
        ],
        out_specs=pl.BlockSpec((TM, DIM), lambda t: (t, 0)),
        out_shape=jax.ShapeDtypeStruct((T, DIM), jnp.float32),
    )(xb, ws1b, ws2b)


def _combine_body(outs_hbm, dflat_hbm, gflat_hbm, sh_hbm, y_hbm,
                  d_v, g_v, d0s_v, d1s_v, r0_v, r1_v, sh_v):
    wid = lax.axis_index("s") * 2 + lax.axis_index("c")
    t0 = wid * (T // _NW)
    lane = lax.iota(jnp.int32, 16)

    @pl.loop(0, T // _NW // 16)
    def _(c):
        tbase = t0 + c * 16
        pltpu.sync_copy(dflat_hbm.at[pl.ds(tbase * 2, 32)], d_v)
        pltpu.sync_copy(gflat_hbm.at[pl.ds(tbase * 2, 32)], g_v)
        d0s_v[...] = plsc.load_gather(d_v, [2 * lane])
        d1s_v[...] = plsc.load_gather(d_v, [2 * lane + 1])
        pltpu.sync_copy(outs_hbm.at[d0s_v], r0_v)
        pltpu.sync_copy(outs_hbm.at[d1s_v], r1_v)
        pltpu.sync_copy(sh_hbm.at[pl.ds(tbase, 16)], sh_v)

        @pl.loop(0, 16)
        def _(i):
            g0 = plsc.load_gather(g_v, [jnp.full((16,), 2 * i, jnp.int32)])
            g1 = plsc.load_gather(g_v, [jnp.full((16,), 2 * i + 1,
                                                 jnp.int32)])

            @pl.loop(0, DIM // 16)
            def _(jj):
                sl = pl.ds(jj * 16, 16)
                r0_v[i, sl] = (g0 * r0_v[i, sl] + g1 * r1_v[i, sl]
                               + sh_v[i, sl])

        pltpu.sync_copy(r0_v, y_hbm.at[pl.ds(tbase, 16)])


@functools.cache
def _combine_kernel():
    return pl.kernel(
        _combine_body,
        out_type=jax.ShapeDtypeStruct((T, DIM), jnp.float32),
        mesh=_sc_mesh(),
        scratch_types=[
            pltpu.VMEM((32,), jnp.int32),
            pltpu.VMEM((32,), jnp.float32),
            pltpu.VMEM((16,), jnp.int32),
            pltpu.VMEM((16,), jnp.int32),
            pltpu.VMEM((16, DIM), jnp.float32),
            pltpu.VMEM((16, DIM), jnp.float32),
            pltpu.VMEM((16, DIM), jnp.float32),
        ],
        compiler_params=_sc_params(),
    )


@jax.jit
def kernel(x, Wg, w1, w2, w3, ws1, ws2):
    orig_shape = x.shape
    xf = x.reshape(-1, orig_shape[-1])
    xb = xf.astype(jnp.bfloat16)
    w1b = w1.astype(jnp.bfloat16)
    w2b = w2.astype(jnp.bfloat16)
    w3b = w3.astype(jnp.bfloat16)
    ws1b = ws1.astype(jnp.bfloat16)
    ws2b = ws2.astype(jnp.bfloat16)

    d2, g2, eid, act = _router(xf, Wg)
    dflat = d2.reshape(A)
    gflat = g2.reshape(A)

    xb_i32 = lax.bitcast_convert_type(xb.reshape(T, DIM // 2, 2), jnp.int32)
    xs_i32 = _dispatch_kernel()(xb_i32, dflat)
    xs = lax.bitcast_convert_type(xs_i32, jnp.bfloat16).reshape(P, DIM)
    sh = _shared(xb, ws1b, ws2b)
    outs = _gmm(xs, w1b, w3b, w2b, eid, act)
    y = _combine_kernel()(outs, dflat, gflat, sh)
    return y.reshape(orig_shape)


# speedup vs baseline: 1.5395x; 1.5395x over previous
import dataclasses
import functools

import jax
import jax.numpy as jnp
from jax import lax
from jax.experimental import pallas as pl
from jax.experimental.pallas import tpu as pltpu
from jax.experimental.pallas import tpu_sc as plsc

DIM = 2048
INTER = 1408
IH = INTER // 2
E = 8
K = 2
T = 2048
A = T * K
MT = 128
P = A + E * MT
NT = P // MT
TT = 4
TM = T // TT

_NW = 32


def _sc_mesh():
    return plsc.VectorSubcoreMesh(core_axis_name="c", subcore_axis_name="s")


def _sc_params():
    cp = pltpu.CompilerParams()
    if "needs_layout_passes" in pltpu.CompilerParams.__dataclass_fields__:
        cp = dataclasses.replace(cp, needs_layout_passes=False)
    return cp


def _dot(a, b):
    return jnp.dot(a, b, precision=lax.Precision.DEFAULT,
                   preferred_element_type=jnp.float32)


def _router_body(x_ref, wg_ref, d_ref, g_ref, eid_ref, act_ref):
    logits = _dot(x_ref[...], wg_ref[...].T)
    eidx = lax.broadcasted_iota(jnp.int32, (T, E), 1)
    m1 = jnp.max(logits, axis=1, keepdims=True)
    e1 = jnp.min(jnp.where(logits >= m1, eidx, E), axis=1, keepdims=True)
    l2 = jnp.where(eidx == e1, -1e30, logits)
    m2 = jnp.max(l2, axis=1, keepdims=True)
    e2 = jnp.min(jnp.where(l2 >= m2, eidx, E), axis=1, keepdims=True)
    q = jnp.exp(m2 - m1)
    wa = 1.0 / (1.0 + q)
    wb = 1.0 - wa
    g_ref[...] = jnp.concatenate([wa, wb], axis=1)

    mask1 = (eidx == e1).astype(jnp.float32)
    mask2 = (eidx == e2).astype(jnp.float32)
    m = mask1 + mask2
    tri = (lax.broadcasted_iota(jnp.int32, (128, 128), 0)
           >= lax.broadcasted_iota(jnp.int32, (128, 128), 1)
           ).astype(jnp.float32)
    cex_blocks = []
    base = jnp.zeros((1, E), jnp.float32)
    for i in range(T // 128):
        mi = m[i * 128:(i + 1) * 128]
        inc = jnp.dot(tri, mi, preferred_element_type=jnp.float32)
        cex_blocks.append(inc - mi + base)
        base = base + inc[127:128, :]
    cex = jnp.concatenate(cex_blocks, axis=0)
    counts = base

    counts_i = counts.astype(jnp.int32)
    padded = ((counts_i + (MT - 1)) // MT) * MT
    stri = (lax.broadcasted_iota(jnp.int32, (E, E), 0)
            < lax.broadcasted_iota(jnp.int32, (E, E), 1)
            ).astype(jnp.float32)
    poffs = jnp.dot(padded.astype(jnp.float32), stri,
                    preferred_element_type=jnp.float32)

    slot = cex + poffs
    d0 = jnp.sum(jnp.where(eidx == e1, slot, 0.0), axis=1, keepdims=True)
    d1 = jnp.sum(jnp.where(eidx == e2, slot, 0.0), axis=1, keepdims=True)
    d_ref[...] = jnp.concatenate([d0, d1], axis=1).astype(jnp.int32)

    pend = poffs + padded.astype(jnp.float32)
    jstart = (lax.broadcasted_iota(jnp.int32, (NT, 1), 0) * MT
              ).astype(jnp.float32)
    eid = jnp.sum((jstart >= pend).astype(jnp.int32), axis=1, keepdims=True)
    eid_ref[...] = jnp.minimum(eid, E - 1)
    act_ref[...] = (jstart < pend[:, E - 1:E]).astype(jnp.int32)


def _router(xf, Wg):
    return pl.pallas_call(
        _router_body,
        out_shape=[
            jax.ShapeDtypeStruct((T, K), jnp.int32),
            jax.ShapeDtypeStruct((T, K), jnp.float32),
            jax.ShapeDtypeStruct((NT, 1), jnp.int32),
            jax.ShapeDtypeStruct((NT, 1), jnp.int32),
        ],
    )(xf, Wg)


def _dispatch_body(xf_hbm, dflat_hbm, xs_hbm, d_v, buf_v):
    wid = lax.axis_index("s") * 2 + lax.axis_index("c")
    a0 = wid * (A // _NW)

    @pl.loop(0, A // _NW // 16)
    def _(c):
        abase = a0 + c * 16
        pltpu.sync_copy(dflat_hbm.at[pl.ds(abase, 16)], d_v)
        tok = (abase + lax.iota(jnp.int32, 16)) >> 1
        pltpu.sync_copy(xf_hbm.at[tok], buf_v)
        pltpu.sync_copy(buf_v, xs_hbm.at[d_v])


@functools.cache
def _dispatch_kernel():
    return pl.kernel(
        _dispatch_body,
        out_type=jax.ShapeDtypeStruct((P, DIM), jnp.float32),
        mesh=_sc_mesh(),
        scratch_types=[
            pltpu.VMEM((16,), jnp.int32),
            pltpu.VMEM((16, DIM), jnp.float32),
        ],
        compiler_params=_sc_params(),
    )


def _gmm_half(h, xs_ref, w1_ref, w3_ref, w2_ref):
    x = xs_ref[...]
    a = _dot(x, w1_ref[0].T)
    b = _dot(x, w3_ref[0].T)
    hh = jax.nn.silu(a) * b
    return _dot(hh, w2_ref[0][:, h * IH:(h + 1) * IH].T)


def _gmm_body_lo(eid_ref, act_ref, xs_ref, w1_ref, w3_ref, w2_ref, o_ref):
    j = pl.program_id(0)

    @pl.when(act_ref[j, 0] == 1)
    def _():
        o_ref[...] = _gmm_half(0, xs_ref, w1_ref, w3_ref, w2_ref)


def _gmm_body_hi(eid_ref, act_ref, xs_ref, w1_ref, w3_ref, w2_ref, olo_ref,
                 o_ref):
    j = pl.program_id(0)

    @pl.when(act_ref[j, 0] == 1)
    def _():
        o_ref[...] = olo_ref[...] + _gmm_half(1, xs_ref, w1_ref, w3_ref,
                                              w2_ref)


def _gmm(xs, w1, w3, w2, eid, act):
    def specs(h, extra):
        return pltpu.PrefetchScalarGridSpec(
            num_scalar_prefetch=2,
            grid=(NT,),
            in_specs=[
                pl.BlockSpec((MT, DIM), lambda j, eid, act: (j, 0)),
                pl.BlockSpec((1, IH, DIM),
                             lambda j, eid, act: (eid[j, 0], h, 0)),
                pl.BlockSpec((1, IH, DIM),
                             lambda j, eid, act: (eid[j, 0], h, 0)),
                pl.BlockSpec((1, DIM, INTER),
                             lambda j, eid, act: (eid[j, 0], 0, 0)),
            ] + extra,
            out_specs=pl.BlockSpec((MT, DIM), lambda j, eid, act: (j, 0)),
        )

    olo = pl.pallas_call(
        _gmm_body_lo,
        grid_spec=specs(0, []),
        out_shape=jax.ShapeDtypeStruct((P, DIM), jnp.float32),
    )(eid, act, xs, w1, w3, w2)
    return pl.pallas_call(
        _gmm_body_hi,
        grid_spec=specs(1, [pl.BlockSpec((MT, DIM),
                                         lambda j, eid, act: (j, 0))]),
        out_shape=jax.ShapeDtypeStruct((P, DIM), jnp.float32),
    )(eid, act, xs, w1, w3, w2, olo)


def _shared_body(x_ref, ws1_ref, ws2_ref, o_ref):
    xb = x_ref[...].astype(jnp.bfloat16)
    a = _dot(xb, ws1_ref[...].T)
    g = (0.5 * a * (1.0 + lax.erf(a * 0.7071067811865476))).astype(
        jnp.bfloat16)
    o_ref[...] = _dot(g, ws2_ref[...].T)


def _shared(xf, ws1b, ws2b):
    return pl.pallas_call(
        _shared_body,
        grid=(TT,),
        in_specs=[
            pl.BlockSpec((TM, DIM), lambda t: (t, 0)),
            pl.BlockSpec((INTER, DIM), lambda t: (0, 0)),
            pl.BlockSpec((DIM, INTER), lambda t: (0, 0)),
        ],
        out_specs=pl.BlockSpec((TM, DIM), lambda t: (t, 0)),
        out_shape=jax.ShapeDtypeStruct((T, DIM), jnp.float32),
    )(xf, ws1b, ws2b)


def _combine_body(outs_hbm, dflat_hbm, gflat_hbm, sh_hbm, y_hbm,
                  d_v, g_v, d0s_v, d1s_v, r0_v, r1_v, sh_v):
    wid = lax.axis_index("s") * 2 + lax.axis_index("c")
    t0 = wid * (T // _NW)
    lane = lax.iota(jnp.int32, 16)

    @pl.loop(0, T // _NW // 16)
    def _(c):
        tbase = t0 + c * 16
        pltpu.sync_copy(dflat_hbm.at[pl.ds(tbase * 2, 32)], d_v)
        pltpu.sync_copy(gflat_hbm.at[pl.ds(tbase * 2, 32)], g_v)
        d0s_v[...] = plsc.load_gather(d_v, [2 * lane])
        d1s_v[...] = plsc.load_gather(d_v, [2 * lane + 1])
        pltpu.sync_copy(outs_hbm.at[d0s_v], r0_v)
        pltpu.sync_copy(outs_hbm.at[d1s_v], r1_v)
        pltpu.sync_copy(sh_hbm.at[pl.ds(tbase, 16)], sh_v)

        @pl.loop(0, 16)
        def _(i):
            g0 = plsc.load_gather(g_v, [jnp.full((16,), 2 * i, jnp.int32)])
            g1 = plsc.load_gather(g_v, [jnp.full((16,), 2 * i + 1,
                                                 jnp.int32)])

            @pl.loop(0, DIM // 16)
            def _(jj):
                sl = pl.ds(jj * 16, 16)
                r0_v[i, sl] = (g0 * r0_v[i, sl] + g1 * r1_v[i, sl]
                               + sh_v[i, sl])

        pltpu.sync_copy(r0_v, y_hbm.at[pl.ds(tbase, 16)])


@functools.cache
def _combine_kernel():
    return pl.kernel(
        _combine_body,
        out_type=jax.ShapeDtypeStruct((T, DIM), jnp.float32),
        mesh=_sc_mesh(),
        scratch_types=[
            pltpu.VMEM((32,), jnp.int32),
            pltpu.VMEM((32,), jnp.float32),
            pltpu.VMEM((16,), jnp.int32),
            pltpu.VMEM((16,), jnp.int32),
            pltpu.VMEM((16, DIM), jnp.float32),
            pltpu.VMEM((16, DIM), jnp.float32),
            pltpu.VMEM((16, DIM), jnp.float32),
        ],
        compiler_params=_sc_params(),
    )


@jax.jit
def kernel(x, Wg, w1, w2, w3, ws1, ws2):
    orig_shape = x.shape
    xf = x.reshape(-1, orig_shape[-1])
    ws1b = ws1.astype(jnp.bfloat16)
    ws2b = ws2.astype(jnp.bfloat16)

    d2, g2, eid, act = _router(xf, Wg)
    dflat = d2.reshape(A)
    gflat = g2.reshape(A)

    xs = _dispatch_kernel()(xf, dflat)
    sh = _shared(xf, ws1b, ws2b)
    outs = _gmm(xs, w1, w3, w2, eid, act)
    y = _combine_kernel()(outs, dflat, gflat, sh)
    return y.reshape(orig_shape)


# speedup vs baseline: 1.6298x; 1.0586x over previous
import dataclasses
import functools

import jax
import jax.numpy as jnp
from jax import lax
from jax.experimental import pallas as pl
from jax.experimental.pallas import tpu as pltpu
from jax.experimental.pallas import tpu_sc as plsc

DIM = 2048
INTER = 1408
IH = INTER // 2
E = 8
K = 2
T = 2048
A = T * K
MT = 128
P = A + E * MT
NT = P // MT
TT = 4
TM = T // TT

_NW = 32


def _sc_mesh():
    return plsc.VectorSubcoreMesh(core_axis_name="c", subcore_axis_name="s")


def _sc_params():
    cp = pltpu.CompilerParams()
    if "needs_layout_passes" in pltpu.CompilerParams.__dataclass_fields__:
        cp = dataclasses.replace(cp, needs_layout_passes=False)
    return cp


def _dot(a, b):
    return jnp.dot(a, b, precision=lax.Precision.DEFAULT,
                   preferred_element_type=jnp.float32)


def _router_body(x_ref, wg_ref, d_ref, g_ref, eid_ref, act_ref):
    logits = _dot(x_ref[...], wg_ref[...].T)
    eidx = lax.broadcasted_iota(jnp.int32, (T, E), 1)
    m1 = jnp.max(logits, axis=1, keepdims=True)
    e1 = jnp.min(jnp.where(logits >= m1, eidx, E), axis=1, keepdims=True)
    l2 = jnp.where(eidx == e1, -1e30, logits)
    m2 = jnp.max(l2, axis=1, keepdims=True)
    e2 = jnp.min(jnp.where(l2 >= m2, eidx, E), axis=1, keepdims=True)
    q = jnp.exp(m2 - m1)
    wa = 1.0 / (1.0 + q)
    wb = 1.0 - wa
    g_ref[...] = jnp.concatenate([wa, wb], axis=1)

    mask1 = (eidx == e1).astype(jnp.float32)
    mask2 = (eidx == e2).astype(jnp.float32)
    m = mask1 + mask2
    tri = (lax.broadcasted_iota(jnp.int32, (128, 128), 0)
           >= lax.broadcasted_iota(jnp.int32, (128, 128), 1)
           ).astype(jnp.float32)
    cex_blocks = []
    base = jnp.zeros((1, E), jnp.float32)
    for i in range(T // 128):
        mi = m[i * 128:(i + 1) * 128]
        inc = jnp.dot(tri, mi, preferred_element_type=jnp.float32)
        cex_blocks.append(inc - mi + base)
        base = base + inc[127:128, :]
    cex = jnp.concatenate(cex_blocks, axis=0)
    counts = base

    counts_i = counts.astype(jnp.int32)
    padded = ((counts_i + (MT - 1)) // MT) * MT
    stri = (lax.broadcasted_iota(jnp.int32, (E, E), 0)
            < lax.broadcasted_iota(jnp.int32, (E, E), 1)
            ).astype(jnp.float32)
    poffs = jnp.dot(padded.astype(jnp.float32), stri,
                    preferred_element_type=jnp.float32)

    slot = cex + poffs
    d0 = jnp.sum(jnp.where(eidx == e1, slot, 0.0), axis=1, keepdims=True)
    d1 = jnp.sum(jnp.where(eidx == e2, slot, 0.0), axis=1, keepdims=True)
    d_ref[...] = jnp.concatenate([d0, d1], axis=1).astype(jnp.int32)

    pend = poffs + padded.astype(jnp.float32)
    jstart = (lax.broadcasted_iota(jnp.int32, (NT, 1), 0) * MT
              ).astype(jnp.float32)
    eid = jnp.sum((jstart >= pend).astype(jnp.int32), axis=1, keepdims=True)
    eid_ref[...] = jnp.minimum(eid, E - 1)
    act_ref[...] = (jstart < pend[:, E - 1:E]).astype(jnp.int32)


def _router(xf, Wg):
    return pl.pallas_call(
        _router_body,
        out_shape=[
            jax.ShapeDtypeStruct((T, K), jnp.int32),
            jax.ShapeDtypeStruct((T, K), jnp.float32),
            jax.ShapeDtypeStruct((NT, 1), jnp.int32),
            jax.ShapeDtypeStruct((NT, 1), jnp.int32),
        ],
    )(xf, Wg)


def _dispatch_body(xf_hbm, dflat_hbm, xs_hbm, d_v, buf_v):
    wid = lax.axis_index("s") * 2 + lax.axis_index("c")
    a0 = wid * (A // _NW)

    @pl.loop(0, A // _NW // 16)
    def _(c):
        abase = a0 + c * 16
        pltpu.sync_copy(dflat_hbm.at[pl.ds(abase, 16)], d_v)
        tok = (abase + lax.iota(jnp.int32, 16)) >> 1
        pltpu.sync_copy(xf_hbm.at[tok], buf_v)
        pltpu.sync_copy(buf_v, xs_hbm.at[d_v])


@functools.cache
def _dispatch_kernel():
    return pl.kernel(
        _dispatch_body,
        out_type=jax.ShapeDtypeStruct((P, DIM), jnp.float32),
        mesh=_sc_mesh(),
        scratch_types=[
            pltpu.VMEM((16,), jnp.int32),
            pltpu.VMEM((16, DIM), jnp.float32),
        ],
        compiler_params=_sc_params(),
    )


def _hlo_body(eid_ref, act_ref, xs_ref, w1_ref, w3_ref, h_ref):
    j = pl.program_id(0)

    @pl.when(act_ref[j, 0] == 1)
    def _():
        x = xs_ref[...]
        a = _dot(x, w1_ref[0].T)
        b = _dot(x, w3_ref[0].T)
        h_ref[...] = jax.nn.silu(a) * b


def _gmm_body(eid_ref, act_ref, xs_ref, w1_ref, w3_ref, hlo_ref, w2_ref,
              o_ref):
    j = pl.program_id(0)

    @pl.when(act_ref[j, 0] == 1)
    def _():
        x = xs_ref[...]
        a = _dot(x, w1_ref[0].T)
        b = _dot(x, w3_ref[0].T)
        h_hi = jax.nn.silu(a) * b
        o_ref[...] = (_dot(hlo_ref[...], w2_ref[0][:, :IH].T)
                      + _dot(h_hi, w2_ref[0][:, IH:].T))


def _gmm(xs, w1, w3, w2, eid, act):
    hlo = pl.pallas_call(
        _hlo_body,
        grid_spec=pltpu.PrefetchScalarGridSpec(
            num_scalar_prefetch=2,
            grid=(NT,),
            in_specs=[
                pl.BlockSpec((MT, DIM), lambda j, eid, act: (j, 0)),
                pl.BlockSpec((1, IH, DIM),
                             lambda j, eid, act: (eid[j, 0], 0, 0)),
                pl.BlockSpec((1, IH, DIM),
                             lambda j, eid, act: (eid[j, 0], 0, 0)),
            ],
            out_specs=pl.BlockSpec((MT, IH), lambda j, eid, act: (j, 0)),
        ),
        out_shape=jax.ShapeDtypeStruct((P, IH), jnp.float32),
    )(eid, act, xs, w1, w3)
    return pl.pallas_call(
        _gmm_body,
        grid_spec=pltpu.PrefetchScalarGridSpec(
            num_scalar_prefetch=2,
            grid=(NT,),
            in_specs=[
                pl.BlockSpec((MT, DIM), lambda j, eid, act: (j, 0)),
                pl.BlockSpec((1, IH, DIM),
                             lambda j, eid, act: (eid[j, 0], 1, 0)),
                pl.BlockSpec((1, IH, DIM),
                             lambda j, eid, act: (eid[j, 0], 1, 0)),
                pl.BlockSpec((MT, IH), lambda j, eid, act: (j, 0)),
                pl.BlockSpec((1, DIM, INTER),
                             lambda j, eid, act: (eid[j, 0], 0, 0)),
            ],
            out_specs=pl.BlockSpec((MT, DIM), lambda j, eid, act: (j, 0)),
        ),
        out_shape=jax.ShapeDtypeStruct((P, DIM), jnp.float32),
    )(eid, act, xs, w1, w3, hlo, w2)


def _shared_body(x_ref, ws1_ref, ws2_ref, o_ref):
    xb = x_ref[...].astype(jnp.bfloat16)
    a = _dot(xb, ws1_ref[...].T)
    g = (0.5 * a * (1.0 + lax.erf(a * 0.7071067811865476))).astype(
        jnp.bfloat16)
    o_ref[...] = _dot(g, ws2_ref[...].T)


def _shared(xf, ws1b, ws2b):
    return pl.pallas_call(
        _shared_body,
        grid=(TT,),
        in_specs=[
            pl.BlockSpec((TM, DIM), lambda t: (t, 0)),
            pl.BlockSpec((INTER, DIM), lambda t: (0, 0)),
            pl.BlockSpec((DIM, INTER), lambda t: (0, 0)),
        ],
        out_specs=pl.BlockSpec((TM, DIM), lambda t: (t, 0)),
        out_shape=jax.ShapeDtypeStruct((T, DIM), jnp.float32),
    )(xf, ws1b, ws2b)


def _combine_body(outs_hbm, dflat_hbm, gflat_hbm, sh_hbm, y_hbm,
                  d_v, g_v, d0s_v, d1s_v, r0_v, r1_v, sh_v):
    wid = lax.axis_index("s") * 2 + lax.axis_index("c")
    t0 = wid * (T // _NW)
    lane = lax.iota(jnp.int32, 16)

    @pl.loop(0, T // _NW // 16)
    def _(c):
        tbase = t0 + c * 16
        pltpu.sync_copy(dflat_hbm.at[pl.ds(tbase * 2, 32)], d_v)
        pltpu.sync_copy(gflat_hbm.at[pl.ds(tbase * 2, 32)], g_v)
        d0s_v[...] = plsc.load_gather(d_v, [2 * lane])
        d1s_v[...] = plsc.load_gather(d_v, [2 * lane + 1])
        pltpu.sync_copy(outs_hbm.at[d0s_v], r0_v)
        pltpu.sync_copy(outs_hbm.at[d1s_v], r1_v)
        pltpu.sync_copy(sh_hbm.at[pl.ds(tbase, 16)], sh_v)

        @pl.loop(0, 16)
        def _(i):
            g0 = plsc.load_gather(g_v, [jnp.full((16,), 2 * i, jnp.int32)])
            g1 = plsc.load_gather(g_v, [jnp.full((16,), 2 * i + 1,
                                                 jnp.int32)])

            @pl.loop(0, DIM // 16)
            def _(jj):
                sl = pl.ds(jj * 16, 16)
                r0_v[i, sl] = (g0 * r0_v[i, sl] + g1 * r1_v[i, sl]
                               + sh_v[i, sl])

        pltpu.sync_copy(r0_v, y_hbm.at[pl.ds(tbase, 16)])


@functools.cache
def _combine_kernel():
    return pl.kernel(
        _combine_body,
        out_type=jax.ShapeDtypeStruct((T, DIM), jnp.float32),
        mesh=_sc_mesh(),
        scratch_types=[
            pltpu.VMEM((32,), jnp.int32),
            pltpu.VMEM((32,), jnp.float32),
            pltpu.VMEM((16,), jnp.int32),
            pltpu.VMEM((16,), jnp.int32),
            pltpu.VMEM((16, DIM), jnp.float32),
            pltpu.VMEM((16, DIM), jnp.float32),
            pltpu.VMEM((16, DIM), jnp.float32),
        ],
        compiler_params=_sc_params(),
    )


@jax.jit
def kernel(x, Wg, w1, w2, w3, ws1, ws2):
    orig_shape = x.shape
    xf = x.reshape(-1, orig_shape[-1])
    ws1b = ws1.astype(jnp.bfloat16)
    ws2b = ws2.astype(jnp.bfloat16)

    d2, g2, eid, act = _router(xf, Wg)
    dflat = d2.reshape(A)
    gflat = g2.reshape(A)

    xs = _dispatch_kernel()(xf, dflat)
    sh = _shared(xf, ws1b, ws2b)
    outs = _gmm(xs, w1, w3, w2, eid, act)
    y = _combine_kernel()(outs, dflat, gflat, sh)
    return y.reshape(orig_shape)


# speedup vs baseline: 1.6400x; 1.0062x over previous
import dataclasses
import functools

import jax
import jax.numpy as jnp
from jax import lax
from jax.experimental import pallas as pl
from jax.experimental.pallas import tpu as pltpu
from jax.experimental.pallas import tpu_sc as plsc

DIM = 2048
INTER = 1408
IH = INTER // 2
E = 8
K = 2
T = 2048
A = T * K
MT = 128
P = A + E * MT
NT = P // MT
TT = 4
TM = T // TT

_NW = 32


def _sc_mesh():
    return plsc.VectorSubcoreMesh(core_axis_name="c", subcore_axis_name="s")


def _sc_params():
    cp = pltpu.CompilerParams()
    if "needs_layout_passes" in pltpu.CompilerParams.__dataclass_fields__:
        cp = dataclasses.replace(cp, needs_layout_passes=False)
    return cp


def _dot(a, b):
    return jnp.dot(a, b, precision=lax.Precision.DEFAULT,
                   preferred_element_type=jnp.float32)


def _router_body(x_ref, wg_ref, d_ref, g_ref, eid_ref, act_ref):
    logits = _dot(x_ref[...], wg_ref[...].T)
    eidx = lax.broadcasted_iota(jnp.int32, (T, E), 1)
    m1 = jnp.max(logits, axis=1, keepdims=True)
    e1 = jnp.min(jnp.where(logits >= m1, eidx, E), axis=1, keepdims=True)
    l2 = jnp.where(eidx == e1, -1e30, logits)
    m2 = jnp.max(l2, axis=1, keepdims=True)
    e2 = jnp.min(jnp.where(l2 >= m2, eidx, E), axis=1, keepdims=True)
    q = jnp.exp(m2 - m1)
    wa = 1.0 / (1.0 + q)
    wb = 1.0 - wa
    g_ref[...] = jnp.concatenate([wa, wb], axis=1)

    mask1 = (eidx == e1).astype(jnp.float32)
    mask2 = (eidx == e2).astype(jnp.float32)
    m = mask1 + mask2
    tri = (lax.broadcasted_iota(jnp.int32, (128, 128), 0)
           >= lax.broadcasted_iota(jnp.int32, (128, 128), 1)
           ).astype(jnp.float32)
    cex_blocks = []
    base = jnp.zeros((1, E), jnp.float32)
    for i in range(T // 128):
        mi = m[i * 128:(i + 1) * 128]
        inc = jnp.dot(tri, mi, preferred_element_type=jnp.float32)
        cex_blocks.append(inc - mi + base)
        base = base + inc[127:128, :]
    cex = jnp.concatenate(cex_blocks, axis=0)
    counts = base

    counts_i = counts.astype(jnp.int32)
    padded = ((counts_i + (MT - 1)) // MT) * MT
    stri = (lax.broadcasted_iota(jnp.int32, (E, E), 0)
            < lax.broadcasted_iota(jnp.int32, (E, E), 1)
            ).astype(jnp.float32)
    poffs = jnp.dot(padded.astype(jnp.float32), stri,
                    preferred_element_type=jnp.float32)

    slot = cex + poffs
    d0 = jnp.sum(jnp.where(eidx == e1, slot, 0.0), axis=1, keepdims=True)
    d1 = jnp.sum(jnp.where(eidx == e2, slot, 0.0), axis=1, keepdims=True)
    d_ref[...] = jnp.concatenate([d0, d1], axis=1).astype(jnp.int32)

    pend = poffs + padded.astype(jnp.float32)
    jstart = (lax.broadcasted_iota(jnp.int32, (NT, 1), 0) * MT
              ).astype(jnp.float32)
    eid = jnp.sum((jstart >= pend).astype(jnp.int32), axis=1, keepdims=True)
    eid_ref[...] = jnp.minimum(eid, E - 1)
    act_ref[...] = (jstart < pend[:, E - 1:E]).astype(jnp.int32)


def _router(xf, Wg):
    return pl.pallas_call(
        _router_body,
        out_shape=[
            jax.ShapeDtypeStruct((T, K), jnp.int32),
            jax.ShapeDtypeStruct((T, K), jnp.float32),
            jax.ShapeDtypeStruct((NT, 1), jnp.int32),
            jax.ShapeDtypeStruct((NT, 1), jnp.int32),
        ],
    )(xf, Wg)


def _dispatch_body(xf_hbm, dflat_hbm, xs_hbm, d_v, buf_v):
    wid = lax.axis_index("s") * 2 + lax.axis_index("c")
    a0 = wid * (A // _NW)

    @pl.loop(0, A // _NW // 16)
    def _(c):
        abase = a0 + c * 16
        pltpu.sync_copy(dflat_hbm.at[pl.ds(abase, 16)], d_v)
        tok = (abase + lax.iota(jnp.int32, 16)) >> 1
        pltpu.sync_copy(xf_hbm.at[tok], buf_v)
        pltpu.sync_copy(buf_v, xs_hbm.at[d_v])


@functools.cache
def _dispatch_kernel():
    return pl.kernel(
        _dispatch_body,
        out_type=jax.ShapeDtypeStruct((P, DIM), jnp.float32),
        mesh=_sc_mesh(),
        scratch_types=[
            pltpu.VMEM((16,), jnp.int32),
            pltpu.VMEM((16, DIM), jnp.float32),
        ],
        compiler_params=_sc_params(),
    )


def _hlo_body(eid_ref, act_ref, xs_ref, w1_ref, w3_ref, h_ref):
    j = pl.program_id(0)

    @pl.when(act_ref[j, 0] == 1)
    def _():
        x = xs_ref[...]
        a = _dot(x, w1_ref[0].T)
        b = _dot(x, w3_ref[0].T)
        h_ref[...] = jax.nn.silu(a) * b


def _gmm_body(eid_ref, act_ref, xs_ref, w1_ref, w3_ref, hlo_ref, w2_ref,
              o_ref):
    j = pl.program_id(0)

    @pl.when(act_ref[j, 0] == 1)
    def _():
        x = xs_ref[...]
        a = _dot(x, w1_ref[0].T)
        b = _dot(x, w3_ref[0].T)
        h_hi = jax.nn.silu(a) * b
        o_ref[...] = (_dot(hlo_ref[...], w2_ref[0][:, :IH].T)
                      + _dot(h_hi, w2_ref[0][:, IH:].T))


def _gmm(xs, w1, w3, w2, eid, act):
    hlo = pl.pallas_call(
        _hlo_body,
        grid_spec=pltpu.PrefetchScalarGridSpec(
            num_scalar_prefetch=2,
            grid=(NT,),
            in_specs=[
                pl.BlockSpec((MT, DIM), lambda j, eid, act: (j, 0)),
                pl.BlockSpec((1, IH, DIM),
                             lambda j, eid, act: (eid[j, 0], 0, 0)),
                pl.BlockSpec((1, IH, DIM),
                             lambda j, eid, act: (eid[j, 0], 0, 0)),
            ],
            out_specs=pl.BlockSpec((MT, IH), lambda j, eid, act: (j, 0)),
        ),
        out_shape=jax.ShapeDtypeStruct((P, IH), jnp.float32),
    )(eid, act, xs, w1, w3)
    return pl.pallas_call(
        _gmm_body,
        grid_spec=pltpu.PrefetchScalarGridSpec(
            num_scalar_prefetch=2,
            grid=(NT,),
            in_specs=[
                pl.BlockSpec((MT, DIM), lambda j, eid, act: (j, 0)),
                pl.BlockSpec((1, IH, DIM),
                             lambda j, eid, act: (eid[j, 0], 1, 0)),
                pl.BlockSpec((1, IH, DIM),
                             lambda j, eid, act: (eid[j, 0], 1, 0)),
                pl.BlockSpec((MT, IH), lambda j, eid, act: (j, 0)),
                pl.BlockSpec((1, DIM, INTER),
                             lambda j, eid, act: (eid[j, 0], 0, 0)),
            ],
            out_specs=pl.BlockSpec((MT, DIM), lambda j, eid, act: (j, 0)),
        ),
        out_shape=jax.ShapeDtypeStruct((P, DIM), jnp.float32),
    )(eid, act, xs, w1, w3, hlo, w2)


def _shared_body(x_ref, ws1_ref, ws2_ref, o_ref):
    xb = x_ref[...].astype(jnp.bfloat16)
    a = _dot(xb, ws1_ref[...].T)
    g = (0.5 * a * (1.0 + lax.erf(a * 0.7071067811865476))).astype(
        jnp.bfloat16)
    o_ref[...] = _dot(g, ws2_ref[...].T)


def _shared(xf, ws1b, ws2b):
    return pl.pallas_call(
        _shared_body,
        grid=(TT,),
        in_specs=[
            pl.BlockSpec((TM, DIM), lambda t: (t, 0)),
            pl.BlockSpec((INTER, DIM), lambda t: (0, 0)),
            pl.BlockSpec((DIM, INTER), lambda t: (0, 0)),
        ],
        out_specs=pl.BlockSpec((TM, DIM), lambda t: (t, 0)),
        out_shape=jax.ShapeDtypeStruct((T, DIM), jnp.float32),
    )(xf, ws1b, ws2b)


def _combine_body(outs_hbm, dflat_hbm, gflat_hbm, sh_hbm, y_hbm,
                  d_v, g_v, d0s_v, d1s_v, r0_v, r1_v, sh_v,
                  sem0, sem1, sem2):
    wid = lax.axis_index("s") * 2 + lax.axis_index("c")
    t0 = wid * (T // _NW)
    lane = lax.iota(jnp.int32, 16)

    @pl.loop(0, T // _NW // 16)
    def _(c):
        tbase = t0 + c * 16
        pltpu.sync_copy(dflat_hbm.at[pl.ds(tbase * 2, 32)], d_v)
        pltpu.sync_copy(gflat_hbm.at[pl.ds(tbase * 2, 32)], g_v)
        d0s_v[...] = plsc.load_gather(d_v, [2 * lane])
        d1s_v[...] = plsc.load_gather(d_v, [2 * lane + 1])
        cp0 = pltpu.async_copy(outs_hbm.at[d0s_v], r0_v, sem0)
        cp1 = pltpu.async_copy(outs_hbm.at[d1s_v], r1_v, sem1)
        cp2 = pltpu.async_copy(sh_hbm.at[pl.ds(tbase, 16)], sh_v, sem2)
        cp0.wait()
        cp1.wait()
        cp2.wait()

        @pl.loop(0, 16)
        def _(i):
            g0 = plsc.load_gather(g_v, [jnp.full((16,), 2 * i, jnp.int32)])
            g1 = plsc.load_gather(g_v, [jnp.full((16,), 2 * i + 1,
                                                 jnp.int32)])

            @pl.loop(0, DIM // 16)
            def _(jj):
                sl = pl.ds(jj * 16, 16)
                r0_v[i, sl] = (g0 * r0_v[i, sl] + g1 * r1_v[i, sl]
                               + sh_v[i, sl])

        pltpu.sync_copy(r0_v, y_hbm.at[pl.ds(tbase, 16)])


@functools.cache
def _combine_kernel():
    return pl.kernel(
        _combine_body,
        out_type=jax.ShapeDtypeStruct((T, DIM), jnp.float32),
        mesh=_sc_mesh(),
        scratch_types=[
            pltpu.VMEM((32,), jnp.int32),
            pltpu.VMEM((32,), jnp.float32),
            pltpu.VMEM((16,), jnp.int32),
            pltpu.VMEM((16,), jnp.int32),
            pltpu.VMEM((16, DIM), jnp.float32),
            pltpu.VMEM((16, DIM), jnp.float32),
            pltpu.VMEM((16, DIM), jnp.float32),
            pltpu.SemaphoreType.DMA,
            pltpu.SemaphoreType.DMA,
            pltpu.SemaphoreType.DMA,
        ],
        compiler_params=_sc_params(),
    )


@jax.jit
def kernel(x, Wg, w1, w2, w3, ws1, ws2):
    orig_shape = x.shape
    xf = x.reshape(-1, orig_shape[-1])
    ws1b = ws1.astype(jnp.bfloat16)
    ws2b = ws2.astype(jnp.bfloat16)

    d2, g2, eid, act = _router(xf, Wg)
    dflat = d2.reshape(A)
    gflat = g2.reshape(A)

    xs = _dispatch_kernel()(xf, dflat)
    sh = _shared(xf, ws1b, ws2b)
    outs = _gmm(xs, w1, w3, w2, eid, act)
    y = _combine_kernel()(outs, dflat, gflat, sh)
    return y.reshape(orig_shape)


# speedup vs baseline: 1.6428x; 1.0017x over previous
import dataclasses
import functools

import jax
import jax.numpy as jnp
from jax import lax
from jax.experimental import pallas as pl
from jax.experimental.pallas import tpu as pltpu
from jax.experimental.pallas import tpu_sc as plsc

DIM = 2048
INTER = 1408
IH = INTER // 2
E = 8
K = 2
T = 2048
A = T * K
MT = 128
P = A + E * MT
NT = P // MT
TT = 4
TM = T // TT

_NW = 32


def _sc_mesh():
    return plsc.VectorSubcoreMesh(core_axis_name="c", subcore_axis_name="s")


def _sc_params():
    cp = pltpu.CompilerParams()
    if "needs_layout_passes" in pltpu.CompilerParams.__dataclass_fields__:
        cp = dataclasses.replace(cp, needs_layout_passes=False)
    return cp


def _dot(a, b):
    return jnp.dot(a, b, precision=lax.Precision.DEFAULT,
                   preferred_element_type=jnp.float32)


def _router_body(x_ref, wg_ref, d_ref, g_ref, eid_ref, act_ref):
    logits = _dot(x_ref[...], wg_ref[...].T)
    eidx = lax.broadcasted_iota(jnp.int32, (T, E), 1)
    m1 = jnp.max(logits, axis=1, keepdims=True)
    e1 = jnp.min(jnp.where(logits >= m1, eidx, E), axis=1, keepdims=True)
    l2 = jnp.where(eidx == e1, -1e30, logits)
    m2 = jnp.max(l2, axis=1, keepdims=True)
    e2 = jnp.min(jnp.where(l2 >= m2, eidx, E), axis=1, keepdims=True)
    q = jnp.exp(m2 - m1)
    wa = 1.0 / (1.0 + q)
    wb = 1.0 - wa
    g_ref[...] = jnp.concatenate([wa, wb], axis=1)

    mask1 = (eidx == e1).astype(jnp.float32)
    mask2 = (eidx == e2).astype(jnp.float32)
    m = mask1 + mask2
    tri = (lax.broadcasted_iota(jnp.int32, (128, 128), 0)
           >= lax.broadcasted_iota(jnp.int32, (128, 128), 1)
           ).astype(jnp.float32)
    cex_blocks = []
    base = jnp.zeros((1, E), jnp.float32)
    for i in range(T // 128):
        mi = m[i * 128:(i + 1) * 128]
        inc = jnp.dot(tri, mi, preferred_element_type=jnp.float32)
        cex_blocks.append(inc - mi + base)
        base = base + inc[127:128, :]
    cex = jnp.concatenate(cex_blocks, axis=0)
    counts = base

    counts_i = counts.astype(jnp.int32)
    padded = ((counts_i + (MT - 1)) // MT) * MT
    stri = (lax.broadcasted_iota(jnp.int32, (E, E), 0)
            < lax.broadcasted_iota(jnp.int32, (E, E), 1)
            ).astype(jnp.float32)
    poffs = jnp.dot(padded.astype(jnp.float32), stri,
                    preferred_element_type=jnp.float32)

    slot = cex + poffs
    d0 = jnp.sum(jnp.where(eidx == e1, slot, 0.0), axis=1, keepdims=True)
    d1 = jnp.sum(jnp.where(eidx == e2, slot, 0.0), axis=1, keepdims=True)
    d_ref[...] = jnp.concatenate([d0, d1], axis=1).astype(jnp.int32)

    pend = poffs + padded.astype(jnp.float32)
    jstart = (lax.broadcasted_iota(jnp.int32, (NT, 1), 0) * MT
              ).astype(jnp.float32)
    eid = jnp.sum((jstart >= pend).astype(jnp.int32), axis=1, keepdims=True)
    eid_ref[...] = jnp.minimum(eid, E - 1)
    act_ref[...] = (jstart < pend[:, E - 1:E]).astype(jnp.int32)


def _router(xf, Wg):
    return pl.pallas_call(
        _router_body,
        out_shape=[
            jax.ShapeDtypeStruct((T, K), jnp.int32),
            jax.ShapeDtypeStruct((T, K), jnp.float32),
            jax.ShapeDtypeStruct((NT, 1), jnp.int32),
            jax.ShapeDtypeStruct((NT, 1), jnp.int32),
        ],
    )(xf, Wg)


def _dispatch_body(xf_hbm, dflat_hbm, xs_hbm, d_v, buf_v):
    wid = lax.axis_index("s") * 2 + lax.axis_index("c")
    a0 = wid * (A // _NW)

    @pl.loop(0, A // _NW // 16)
    def _(c):
        abase = a0 + c * 16
        pltpu.sync_copy(dflat_hbm.at[pl.ds(abase, 16)], d_v)
        tok = (abase + lax.iota(jnp.int32, 16)) >> 1
        pltpu.sync_copy(xf_hbm.at[tok], buf_v)
        pltpu.sync_copy(buf_v, xs_hbm.at[d_v])


@functools.cache
def _dispatch_kernel():
    return pl.kernel(
        _dispatch_body,
        out_type=jax.ShapeDtypeStruct((P, DIM), jnp.float32),
        mesh=_sc_mesh(),
        scratch_types=[
            pltpu.VMEM((16,), jnp.int32),
            pltpu.VMEM((16, DIM), jnp.float32),
        ],
        compiler_params=_sc_params(),
    )


def _hlo_body(eid_ref, act_ref, xs_ref, w1_ref, w3_ref, h_ref):
    j = pl.program_id(0)

    @pl.when(act_ref[j, 0] == 1)
    def _():
        x = xs_ref[...]
        a = _dot(x, w1_ref[0].T)
        b = _dot(x, w3_ref[0].T)
        h_ref[...] = (jax.nn.silu(a) * b).astype(jnp.bfloat16)


def _gmm_body(eid_ref, act_ref, xs_ref, w1_ref, w3_ref, hlo_ref, w2_ref,
              o_ref):
    j = pl.program_id(0)

    @pl.when(act_ref[j, 0] == 1)
    def _():
        x = xs_ref[...]
        a = _dot(x, w1_ref[0].T)
        b = _dot(x, w3_ref[0].T)
        h_hi = jax.nn.silu(a) * b
        o_ref[...] = (_dot(hlo_ref[...].astype(jnp.float32),
                           w2_ref[0][:, :IH].T)
                      + _dot(h_hi, w2_ref[0][:, IH:].T))


def _gmm(xs, w1, w3, w2, eid, act):
    hlo = pl.pallas_call(
        _hlo_body,
        grid_spec=pltpu.PrefetchScalarGridSpec(
            num_scalar_prefetch=2,
            grid=(NT,),
            in_specs=[
                pl.BlockSpec((MT, DIM), lambda j, eid, act: (j, 0)),
                pl.BlockSpec((1, IH, DIM),
                             lambda j, eid, act: (eid[j, 0], 0, 0)),
                pl.BlockSpec((1, IH, DIM),
                             lambda j, eid, act: (eid[j, 0], 0, 0)),
            ],
            out_specs=pl.BlockSpec((MT, IH), lambda j, eid, act: (j, 0)),
        ),
        out_shape=jax.ShapeDtypeStruct((P, IH), jnp.bfloat16),
    )(eid, act, xs, w1, w3)
    return pl.pallas_call(
        _gmm_body,
        grid_spec=pltpu.PrefetchScalarGridSpec(
            num_scalar_prefetch=2,
            grid=(NT,),
            in_specs=[
                pl.BlockSpec((MT, DIM), lambda j, eid, act: (j, 0)),
                pl.BlockSpec((1, IH, DIM),
                             lambda j, eid, act: (eid[j, 0], 1, 0)),
                pl.BlockSpec((1, IH, DIM),
                             lambda j, eid, act: (eid[j, 0], 1, 0)),
                pl.BlockSpec((MT, IH), lambda j, eid, act: (j, 0)),
                pl.BlockSpec((1, DIM, INTER),
                             lambda j, eid, act: (eid[j, 0], 0, 0)),
            ],
            out_specs=pl.BlockSpec((MT, DIM), lambda j, eid, act: (j, 0)),
        ),
        out_shape=jax.ShapeDtypeStruct((P, DIM), jnp.float32),
    )(eid, act, xs, w1, w3, hlo, w2)


def _shared_body(x_ref, ws1_ref, ws2_ref, o_ref):
    xb = x_ref[...].astype(jnp.bfloat16)
    a = _dot(xb, ws1_ref[...].T)
    g = (0.5 * a * (1.0 + lax.erf(a * 0.7071067811865476))).astype(
        jnp.bfloat16)
    o_ref[...] = _dot(g, ws2_ref[...].T)


def _shared(xf, ws1b, ws2b):
    return pl.pallas_call(
        _shared_body,
        grid=(TT,),
        in_specs=[
            pl.BlockSpec((TM, DIM), lambda t: (t, 0)),
            pl.BlockSpec((INTER, DIM), lambda t: (0, 0)),
            pl.BlockSpec((DIM, INTER), lambda t: (0, 0)),
        ],
        out_specs=pl.BlockSpec((TM, DIM), lambda t: (t, 0)),
        out_shape=jax.ShapeDtypeStruct((T, DIM), jnp.float32),
    )(xf, ws1b, ws2b)


def _combine_body(outs_hbm, dflat_hbm, gflat_hbm, sh_hbm, y_hbm,
                  d_v, g_v, d0s_v, d1s_v, r0_v, r1_v, sh_v,
                  sem0, sem1, sem2):
    wid = lax.axis_index("s") * 2 + lax.axis_index("c")
    t0 = wid * (T // _NW)
    lane = lax.iota(jnp.int32, 16)

    @pl.loop(0, T // _NW // 16)
    def _(c):
        tbase = t0 + c * 16
        pltpu.sync_copy(dflat_hbm.at[pl.ds(tbase * 2, 32)], d_v)
        pltpu.sync_copy(gflat_hbm.at[pl.ds(tbase * 2, 32)], g_v)
        d0s_v[...] = plsc.load_gather(d_v, [2 * lane])
        d1s_v[...] = plsc.load_gather(d_v, [2 * lane + 1])
        cp0 = pltpu.async_copy(outs_hbm.at[d0s_v], r0_v, sem0)
        cp1 = pltpu.async_copy(outs_hbm.at[d1s_v], r1_v, sem1)
        cp2 = pltpu.async_copy(sh_hbm.at[pl.ds(tbase, 16)], sh_v, sem2)
        cp0.wait()
        cp1.wait()
        cp2.wait()

        @pl.loop(0, 16)
        def _(i):
            g0 = plsc.load_gather(g_v, [jnp.full((16,), 2 * i, jnp.int32)])
            g1 = plsc.load_gather(g_v, [jnp.full((16,), 2 * i + 1,
                                                 jnp.int32)])

            @pl.loop(0, DIM // 16)
            def _(jj):
                sl = pl.ds(jj * 16, 16)
                r0_v[i, sl] = (g0 * r0_v[i, sl] + g1 * r1_v[i, sl]
                               + sh_v[i, sl])

        pltpu.sync_copy(r0_v, y_hbm.at[pl.ds(tbase, 16)])


@functools.cache
def _combine_kernel():
    return pl.kernel(
        _combine_body,
        out_type=jax.ShapeDtypeStruct((T, DIM), jnp.float32),
        mesh=_sc_mesh(),
        scratch_types=[
            pltpu.VMEM((32,), jnp.int32),
            pltpu.VMEM((32,), jnp.float32),
            pltpu.VMEM((16,), jnp.int32),
            pltpu.VMEM((16,), jnp.int32),
            pltpu.VMEM((16, DIM), jnp.float32),
            pltpu.VMEM((16, DIM), jnp.float32),
            pltpu.VMEM((16, DIM), jnp.float32),
            pltpu.SemaphoreType.DMA,
            pltpu.SemaphoreType.DMA,
            pltpu.SemaphoreType.DMA,
        ],
        compiler_params=_sc_params(),
    )


@jax.jit
def kernel(x, Wg, w1, w2, w3, ws1, ws2):
    orig_shape = x.shape
    xf = x.reshape(-1, orig_shape[-1])
    ws1b = ws1.astype(jnp.bfloat16)
    ws2b = ws2.astype(jnp.bfloat16)

    d2, g2, eid, act = _router(xf, Wg)
    dflat = d2.reshape(A)
    gflat = g2.reshape(A)

    xs = _dispatch_kernel()(xf, dflat)
    sh = _shared(xf, ws1b, ws2b)
    outs = _gmm(xs, w1, w3, w2, eid, act)
    y = _combine_kernel()(outs, dflat, gflat, sh)
    return y.reshape(orig_shape)
